# Initial kernel scaffold; baseline (speedup 1.0000x reference)
import jax
import jax.numpy as jnp
from jax.experimental import pallas as pl


def kernel(emb, edge_index, edge_attr, W1, b1, W2, b2, We1, be1, Wg1, bg1, We2, be2, Wg2, bg2, Wn1, bn1, Wn2, bn2):
    raise NotImplementedError("write your pallas kernel here")


# speedup vs baseline: 2.7048x; 2.7048x over previous
import functools

import jax
import jax.numpy as jnp
from jax import lax
from jax.experimental import pallas as pl
from jax.experimental.pallas import tpu as pltpu
from jax.experimental.pallas import tpu_sc as plsc

_N = 10000
_E = 320000
_D = 128
_NPAD = 10240
_C = 128
_NCHUNK = _E // _C
_NC = 2
_NS = 16
_NW = _NC * _NS
_ROWS_PER_TILE = _NPAD // _NS
_LANES = 16


def _sc_edge_body(x_hbm, src_hbm, dst_hbm, e_hbm, out_hbm,
                  src_v, dst_v, e_v, xs_v, agg_sh, sem):
    c = lax.axis_index("c")
    s = lax.axis_index("s")
    wid = c * _NS + s

    zero16 = jnp.zeros((_LANES,), jnp.float32)

    def _zero_row(i, carry):
        for j in range(_D // _LANES):
            e_v[i, pl.ds(j * _LANES, _LANES)] = zero16
        return carry

    lax.fori_loop(0, _C, _zero_row, 0)
    row0 = s * _ROWS_PER_TILE
    for k in range(_ROWS_PER_TILE // _C):
        pltpu.sync_copy(e_v, agg_sh.at[pl.ds(row0 + k * _C, _C)])
    plsc.subcore_barrier()

    nfull = _NCHUNK // _NW
    rem = _NCHUNK - nfull * _NW
    nk = nfull + jnp.where(wid < rem, 1, 0)

    def chunk_body(k, carry):
        g = wid + k * _NW
        base = pl.multiple_of(g * _C, _C)
        pltpu.sync_copy(src_hbm.at[pl.ds(base, _C)], src_v)
        pltpu.sync_copy(dst_hbm.at[pl.ds(base, _C)], dst_v)
        pltpu.sync_copy(e_hbm.at[pl.ds(base, _C)], e_v)
        pltpu.async_copy(x_hbm.at[src_v], xs_v, sem).wait()

        def row_body(i, rc):
            for j in range(_D // _LANES):
                sl = pl.ds(j * _LANES, _LANES)
                e_v[i, sl] = jnp.maximum(xs_v[i, sl] + e_v[i, sl], 0.0)
            return rc

        lax.fori_loop(0, _C, row_body, 0)
        pltpu.sync_copy(e_v, agg_sh.at[dst_v], add=True)
        return carry

    lax.fori_loop(0, nk, chunk_body, 0)

    plsc.subcore_barrier()
    pltpu.sync_copy(agg_sh.at[pl.ds(row0, _ROWS_PER_TILE)],
                    out_hbm.at[c, pl.ds(row0, _ROWS_PER_TILE)])


@functools.cache
def _build_sc_edge_layer():
    return pl.kernel(
        _sc_edge_body,
        out_type=jax.ShapeDtypeStruct((_NC, _NPAD, _D), jnp.float32),
        mesh=plsc.VectorSubcoreMesh(core_axis_name="c",
                                    subcore_axis_name="s",
                                    num_cores=_NC, num_subcores=_NS),
        scratch_types=[
            pltpu.VMEM((_C,), jnp.int32),
            pltpu.VMEM((_C,), jnp.int32),
            pltpu.VMEM((_C, _D), jnp.float32),
            pltpu.VMEM((_C, _D), jnp.float32),
            pltpu.VMEM_SHARED((_NPAD, _D), jnp.float32),
            pltpu.SemaphoreType.DMA,
        ],
    )


def _sc_edge_layer(x, src, dst, e):
    return _build_sc_edge_layer()(x, src, dst, e)


def _mlp_body(emb_ref, w1_ref, b1_ref, w2_ref, b2_ref, o_ref):
    h = jnp.maximum(
        jnp.dot(emb_ref[...], w1_ref[...],
                preferred_element_type=jnp.float32) + b1_ref[...], 0.0)
    o_ref[...] = jnp.dot(h, w2_ref[...],
                         preferred_element_type=jnp.float32) + b2_ref[...]


def _decoder_mlp(emb, W1, b1, W2, b2):
    blk = 2560
    grid = W2.shape[1] // blk
    return pl.pallas_call(
        _mlp_body,
        grid=(grid,),
        in_specs=[
            pl.BlockSpec((emb.shape[0], emb.shape[1]), lambda j: (0, 0)),
            pl.BlockSpec((W1.shape[0], W1.shape[1]), lambda j: (0, 0)),
            pl.BlockSpec((1, b1.shape[0]), lambda j: (0, 0)),
            pl.BlockSpec((W2.shape[0], blk), lambda j: (0, j)),
            pl.BlockSpec((1, blk), lambda j: (0, j)),
        ],
        out_specs=pl.BlockSpec((emb.shape[0], blk), lambda j: (0, j)),
        out_shape=jax.ShapeDtypeStruct((emb.shape[0], W2.shape[1]),
                                       jnp.float32),
    )(emb, W1, b1.reshape(1, -1), W2, b2.reshape(1, -1))


def _edge_body(ea_ref, we1_ref, be1_ref, we2_ref, be2_ref, e1_ref, e2_ref):
    a = ea_ref[...]
    e1_ref[...] = jnp.dot(a, we1_ref[...],
                          preferred_element_type=jnp.float32) + be1_ref[...]
    e2_ref[...] = jnp.dot(a, we2_ref[...],
                          preferred_element_type=jnp.float32) + be2_ref[...]


def _edge_embed(edge_attr, We1, be1, We2, be2):
    eblk = 10000
    grid = _E // eblk
    de = edge_attr.shape[1]
    espec = pl.BlockSpec((eblk, _D), lambda j: (j, 0))
    return pl.pallas_call(
        _edge_body,
        grid=(grid,),
        in_specs=[
            pl.BlockSpec((eblk, de), lambda j: (j, 0)),
            pl.BlockSpec((de, _D), lambda j: (0, 0)),
            pl.BlockSpec((1, _D), lambda j: (0, 0)),
            pl.BlockSpec((de, _D), lambda j: (0, 0)),
            pl.BlockSpec((1, _D), lambda j: (0, 0)),
        ],
        out_specs=[espec, espec],
        out_shape=[jax.ShapeDtypeStruct((_E, _D), jnp.float32),
                   jax.ShapeDtypeStruct((_E, _D), jnp.float32)],
    )(edge_attr, We1, be1.reshape(1, -1), We2, be2.reshape(1, -1))


def _update_body(x_ref, a_ref, wg_ref, bg_ref, o_ref):
    t = x_ref[...] + a_ref[0] + a_ref[1]
    o_ref[...] = jnp.maximum(
        jnp.dot(t, wg_ref[...], preferred_element_type=jnp.float32)
        + bg_ref[...], 0.0)


def _node_update(x, aggp, Wg, bg):
    return pl.pallas_call(
        _update_body,
        grid=(1,),
        in_specs=[
            pl.BlockSpec((_N, _D), lambda i: (0, 0)),
            pl.BlockSpec((_NC, _N, _D), lambda i: (0, 0, 0)),
            pl.BlockSpec((_D, _D), lambda i: (0, 0)),
            pl.BlockSpec((1, _D), lambda i: (0, 0)),
        ],
        out_specs=pl.BlockSpec((_N, _D), lambda i: (0, 0)),
        out_shape=jax.ShapeDtypeStruct((_N, _D), jnp.float32),
    )(x, aggp, Wg, bg.reshape(1, -1))


def _final_body(x_ref, a_ref, wg_ref, bg_ref, wn1_ref, bn1_ref,
                wn2_ref, bn2_ref, o_ref):
    t = x_ref[...] + a_ref[0] + a_ref[1]
    x = jnp.maximum(
        jnp.dot(t, wg_ref[...], preferred_element_type=jnp.float32)
        + bg_ref[...], 0.0)
    h = jnp.maximum(
        jnp.dot(x, wn1_ref[...], preferred_element_type=jnp.float32)
        + bn1_ref[...], 0.0)
    o_ref[...] = jnp.dot(h, wn2_ref[...],
                         preferred_element_type=jnp.float32) + bn2_ref[...]


def _final_stage(x, aggp, Wg, bg, Wn1, bn1, Wn2, bn2):
    full = lambda *shape: pl.BlockSpec(shape, lambda i: (0,) * len(shape))
    return pl.pallas_call(
        _final_body,
        grid=(1,),
        in_specs=[
            full(_N, _D), full(_NC, _N, _D), full(_D, _D), full(1, _D),
            full(_D, _D), full(1, _D), full(_D, _D), full(1, _D),
        ],
        out_specs=full(_N, _D),
        out_shape=jax.ShapeDtypeStruct((_N, _D), jnp.float32),
    )(x, aggp, Wg, bg.reshape(1, -1), Wn1, bn1.reshape(1, -1),
      Wn2, bn2.reshape(1, -1))


def kernel(emb, edge_index, edge_attr, W1, b1, W2, b2, We1, be1, Wg1, bg1,
           We2, be2, Wg2, bg2, Wn1, bn1, Wn2, bn2):
    src = edge_index[0]
    dst = edge_index[1]
    x0 = _decoder_mlp(emb, W1, b1, W2, b2)
    x = x0.reshape(_N, _D)
    e1, e2 = _edge_embed(edge_attr, We1, be1, We2, be2)
    agg1 = _sc_edge_layer(x, src, dst, e1)
    x = _node_update(x, agg1, Wg1, bg1)
    agg2 = _sc_edge_layer(x, src, dst, e2)
    return _final_stage(x, agg2, Wg2, bg2, Wn1, bn1, Wn2, bn2)
